# Initial kernel scaffold; baseline (speedup 1.0000x reference)
import jax
import jax.numpy as jnp
from jax.experimental import pallas as pl


def kernel(x, edge_index, W1, b1, W2, b2, W3, b3, W4, b4):
    raise NotImplementedError("write your pallas kernel here")


# speedup vs baseline: 10.8050x; 10.8050x over previous
import functools

import jax
import jax.numpy as jnp
from jax import lax
from jax.experimental import pallas as pl
from jax.experimental.pallas import tpu as pltpu
from jax.experimental.pallas import tpu_sc as plsc

N = 10000
E = 320000
NPAD = 10240

NC = 2
NS = 16
NW = NC * NS
EP = E // NW
CHUNK = 80
NCHUNK = EP // CHUNK
ROWS_PER_TILE = NPAD // NS

_MESH = plsc.VectorSubcoreMesh(core_axis_name="c", subcore_axis_name="s")


@functools.partial(
    pl.kernel,
    out_type=jax.ShapeDtypeStruct((NC, NPAD), jnp.float32),
    mesh=_MESH,
    scratch_types=[
        pltpu.VMEM((CHUNK,), jnp.int32),
        pltpu.VMEM((CHUNK,), jnp.float32),
        pltpu.VMEM((ROWS_PER_TILE,), jnp.float32),
        pltpu.VMEM_SHARED((NPAD,), jnp.float32),
    ],
)
def _deg_kernel(dst_hbm, out_hbm, dst_idx, ones_v, zbuf, acc):
    c = lax.axis_index("c")
    s = lax.axis_index("s")
    wid = s * NC + c

    def fill(i, _):
        ones_v[pl.ds(i * 16, 16)] = jnp.ones((16,), jnp.float32)
        return 0

    lax.fori_loop(0, CHUNK // 16, fill, 0)

    def zfill(i, _):
        zbuf[pl.ds(i * 16, 16)] = jnp.zeros((16,), jnp.float32)
        return 0

    lax.fori_loop(0, ROWS_PER_TILE // 16, zfill, 0)
    pltpu.sync_copy(zbuf, acc.at[pl.ds(s * ROWS_PER_TILE, ROWS_PER_TILE)])
    plsc.subcore_barrier()

    def body(i, _):
        base = wid * EP + i * CHUNK
        pltpu.sync_copy(dst_hbm.at[pl.ds(base, CHUNK)], dst_idx)
        pltpu.sync_copy(ones_v, acc.at[dst_idx], add=True)
        return 0

    lax.fori_loop(0, NCHUNK, body, 0)
    plsc.subcore_barrier()
    pltpu.sync_copy(
        acc.at[pl.ds(s * ROWS_PER_TILE, ROWS_PER_TILE)],
        out_hbm.at[c, pl.ds(s * ROWS_PER_TILE, ROWS_PER_TILE)],
    )


def _make_sc_apply(D):
    @functools.partial(
        pl.kernel,
        out_type=jax.ShapeDtypeStruct((NC, NPAD, D), jnp.float32),
        mesh=_MESH,
        scratch_types=[
            pltpu.VMEM((CHUNK,), jnp.int32),
            pltpu.VMEM((CHUNK,), jnp.int32),
            pltpu.VMEM((CHUNK, D), jnp.float32),
            pltpu.VMEM((CHUNK, D), jnp.float32),
            pltpu.VMEM_SHARED((NPAD, D), jnp.float32),
            pltpu.SemaphoreType.DMA,
        ],
        compiler_params=pltpu.CompilerParams(use_tc_tiling_on_sc=False),
    )
    def sc_apply(p_hbm, src_hbm, dst_hbm, out_hbm, src_idx, dst_idx, rows,
                 zbuf, acc, gsem):
        c = lax.axis_index("c")
        s = lax.axis_index("s")
        wid = s * NC + c

        nz = (CHUNK * D) // 16

        def zfill(i, _):
            r = i // (D // 16)
            col = (i % (D // 16)) * 16
            zbuf[r, pl.ds(col, 16)] = jnp.zeros((16,), jnp.float32)
            return 0

        lax.fori_loop(0, nz, zfill, 0)

        def zcopy(i, _):
            pltpu.sync_copy(
                zbuf, acc.at[pl.ds(s * ROWS_PER_TILE + i * CHUNK, CHUNK)]
            )
            return 0

        lax.fori_loop(0, ROWS_PER_TILE // CHUNK, zcopy, 0)
        plsc.subcore_barrier()

        def body(i, _):
            base = wid * EP + i * CHUNK
            pltpu.sync_copy(src_hbm.at[pl.ds(base, CHUNK)], src_idx)
            pltpu.sync_copy(dst_hbm.at[pl.ds(base, CHUNK)], dst_idx)
            pltpu.async_copy(p_hbm.at[src_idx], rows, gsem).wait()
            pltpu.sync_copy(rows, acc.at[dst_idx], add=True)
            return 0

        lax.fori_loop(0, NCHUNK, body, 0)
        plsc.subcore_barrier()
        pltpu.sync_copy(
            acc.at[pl.ds(s * ROWS_PER_TILE, ROWS_PER_TILE)],
            out_hbm.at[c, pl.ds(s * ROWS_PER_TILE, ROWS_PER_TILE)],
        )

    return sc_apply


_sc_apply = {D: _make_sc_apply(D) for D in (128, 64, 32)}


_RB = 1280
_GRID = NPAD // _RB


def _tc0_body(x_ref, w_ref, d0_ref, d1_ref, p_ref, dis_ref):
    dis = lax.rsqrt(d0_ref[...] + d1_ref[...] + 1.0)
    h = jnp.dot(x_ref[...], w_ref[...], preferred_element_type=jnp.float32)
    p_ref[...] = dis * h
    dis_ref[...] = dis


def _tc0(xpad, W1, deg0, deg1):
    D = W1.shape[1]
    return pl.pallas_call(
        _tc0_body,
        grid=(_GRID,),
        in_specs=[
            pl.BlockSpec((_RB, xpad.shape[1]), lambda i: (i, 0)),
            pl.BlockSpec(W1.shape, lambda i: (0, 0)),
            pl.BlockSpec((_RB, 1), lambda i: (i, 0)),
            pl.BlockSpec((_RB, 1), lambda i: (i, 0)),
        ],
        out_specs=[
            pl.BlockSpec((_RB, D), lambda i: (i, 0)),
            pl.BlockSpec((_RB, 1), lambda i: (i, 0)),
        ],
        out_shape=[
            jax.ShapeDtypeStruct((NPAD, D), jnp.float32),
            jax.ShapeDtypeStruct((NPAD, 1), jnp.float32),
        ],
    )(xpad, W1, deg0, deg1)


def _tc_layer_body(s0_ref, s1_ref, p_ref, dis_ref, b_ref, w_ref, out_ref):
    dis = dis_ref[...]
    a = dis * (s0_ref[...] + s1_ref[...] + p_ref[...]) + b_ref[...]
    h = jnp.maximum(a, 0.0)
    out_ref[...] = dis * jnp.dot(
        h, w_ref[...], preferred_element_type=jnp.float32
    )


def _tc_layer(s0, s1, p, dis, b, Wn):
    Din, Dout = Wn.shape
    return pl.pallas_call(
        _tc_layer_body,
        grid=(_GRID,),
        in_specs=[
            pl.BlockSpec((_RB, Din), lambda i: (i, 0)),
            pl.BlockSpec((_RB, Din), lambda i: (i, 0)),
            pl.BlockSpec((_RB, Din), lambda i: (i, 0)),
            pl.BlockSpec((_RB, 1), lambda i: (i, 0)),
            pl.BlockSpec((1, Din), lambda i: (0, 0)),
            pl.BlockSpec((Din, Dout), lambda i: (0, 0)),
        ],
        out_specs=pl.BlockSpec((_RB, Dout), lambda i: (i, 0)),
        out_shape=jax.ShapeDtypeStruct((NPAD, Dout), jnp.float32),
    )(s0, s1, p, dis, b.reshape(1, Din), Wn)


def _tc_final_body(s0_ref, s1_ref, p_ref, dis_ref, b_ref, out_ref):
    out_ref[...] = (
        dis_ref[...] * (s0_ref[...] + s1_ref[...] + p_ref[...]) + b_ref[...]
    )


def _tc_final(s0, s1, p, dis, b):
    D = p.shape[1]
    return pl.pallas_call(
        _tc_final_body,
        grid=(_GRID,),
        in_specs=[
            pl.BlockSpec((_RB, D), lambda i: (i, 0)),
            pl.BlockSpec((_RB, D), lambda i: (i, 0)),
            pl.BlockSpec((_RB, D), lambda i: (i, 0)),
            pl.BlockSpec((_RB, 1), lambda i: (i, 0)),
            pl.BlockSpec((1, D), lambda i: (0, 0)),
        ],
        out_specs=pl.BlockSpec((_RB, D), lambda i: (i, 0)),
        out_shape=jax.ShapeDtypeStruct((NPAD, D), jnp.float32),
    )(s0, s1, p, dis, b.reshape(1, D))


def kernel(x, edge_index, W1, b1, W2, b2, W3, b3, W4, b4):
    src = edge_index[0]
    dst = edge_index[1]
    xpad = jnp.pad(x, ((0, NPAD - N), (0, 0)))

    degp = _deg_kernel(dst)
    deg0 = degp[0].reshape(NPAD, 1)
    deg1 = degp[1].reshape(NPAD, 1)

    p1, dis = _tc0(xpad, W1, deg0, deg1)
    s1 = _sc_apply[128](p1, src, dst)
    p2 = _tc_layer(s1[0], s1[1], p1, dis, b1, W2)
    s2 = _sc_apply[128](p2, src, dst)
    p3 = _tc_layer(s2[0], s2[1], p2, dis, b2, W3)
    s3 = _sc_apply[64](p3, src, dst)
    p4 = _tc_layer(s3[0], s3[1], p3, dis, b3, W4)
    s4 = _sc_apply[32](p4, src, dst)
    z = _tc_final(s4[0], s4[1], p4, dis, b4)
    return z[:N]
